# baseline (speedup 1.0000x reference)
import functools

import jax
import jax.numpy as jnp
from jax import lax
from jax.experimental import pallas as pl
from jax.experimental.pallas import tpu as pltpu
from jax.experimental.pallas import tpu_sc as plsc

T_TILE = 2048
NEG_INF = float("-inf")


def _mlp_body(x_ref, w1_ref, b1_ref, w2_ref, b2_ref, wg_ref,
              logits_ref, feat_ref):
    h = jnp.maximum(
        jnp.dot(x_ref[...], w1_ref[...], preferred_element_type=jnp.float32)
        + b1_ref[...], 0.0)
    h = jnp.maximum(
        jnp.dot(h, w2_ref[...], preferred_element_type=jnp.float32)
        + b2_ref[...], 0.0)
    feat_ref[...] = h
    logits_ref[...] = jax.lax.dot_general(
        wg_ref[...], h, (((0,), (1,)), ((), ())),
        preferred_element_type=jnp.float32)


def _make_sc_router(ntok, n_e):
    info = plsc.get_sparse_core_info()
    nw = info.num_cores * info.num_subcores
    tok_w = ntok // nw
    lanes = info.num_lanes
    mesh = plsc.VectorSubcoreMesh(core_axis_name="c", subcore_axis_name="s")

    @functools.partial(
        pl.kernel, mesh=mesh,
        out_type=(
            jax.ShapeDtypeStruct((n_e, ntok), jnp.float32),
            jax.ShapeDtypeStruct((2, ntok), jnp.int32),
            jax.ShapeDtypeStruct((2, ntok), jnp.float32),
        ),
        scratch_types=[
            pltpu.VMEM((n_e, tok_w), jnp.float32),
            pltpu.VMEM((n_e, tok_w), jnp.float32),
            pltpu.VMEM((2, tok_w), jnp.int32),
            pltpu.VMEM((2, tok_w), jnp.float32),
        ],
    )
    def sc_router(logt_hbm, probs_hbm, tki_hbm, tkp_hbm,
                  lg_v, pr_v, ki_v, kp_v):
        wid = lax.axis_index("s") * info.num_cores + lax.axis_index("c")
        base = wid * tok_w
        pltpu.sync_copy(logt_hbm.at[:, pl.ds(base, tok_w)], lg_v)

        def group(g, carry):
            off = g * lanes
            v = [lg_v[e, pl.ds(off, lanes)] for e in range(n_e)]
            m1 = v[0]
            i1 = jnp.zeros((lanes,), jnp.int32)
            for e in range(1, n_e):
                gt = v[e] > m1
                m1 = jnp.where(gt, v[e], m1)
                i1 = jnp.where(gt, e, i1)
            s = jnp.zeros((lanes,), jnp.float32)
            ex = []
            for e in range(n_e):
                ee = jnp.exp(v[e] - m1)
                ex.append(ee)
                s = s + ee
            r = 1.0 / s
            for e in range(n_e):
                pr_v[e, pl.ds(off, lanes)] = ex[e] * r
            m2 = jnp.full((lanes,), NEG_INF, jnp.float32)
            i2 = jnp.zeros((lanes,), jnp.int32)
            for e in range(n_e):
                gt = (v[e] > m2) & (i1 != e)
                m2 = jnp.where(gt, v[e], m2)
                i2 = jnp.where(gt, e, i2)
            ki_v[0, pl.ds(off, lanes)] = i1
            ki_v[1, pl.ds(off, lanes)] = i2
            p1 = r
            p2 = jnp.exp(m2 - m1) * r
            dn = 1.0 / (p1 + p2)
            kp_v[0, pl.ds(off, lanes)] = p1 * dn
            kp_v[1, pl.ds(off, lanes)] = p2 * dn
            return carry

        lax.fori_loop(0, tok_w // lanes, group, 0, unroll=2)
        pltpu.sync_copy(pr_v, probs_hbm.at[:, pl.ds(base, tok_w)])
        pltpu.sync_copy(ki_v, tki_hbm.at[:, pl.ds(base, tok_w)])
        pltpu.sync_copy(kp_v, tkp_hbm.at[:, pl.ds(base, tok_w)])

    return sc_router


@jax.jit
def kernel(x, W1, b1, W2, b2, Wg):
    ntok, hidden = x.shape
    rhid = W1.shape[1]
    n_e = Wg.shape[1]
    grid = ntok // T_TILE

    tok_spec = lambda w: pl.BlockSpec((T_TILE, w), lambda i: (i, 0))
    tr_spec = lambda rows: pl.BlockSpec((rows, T_TILE), lambda i: (0, i))
    fixed_spec = lambda a, b: pl.BlockSpec((a, b), lambda i: (0, 0))

    lt, feat = pl.pallas_call(
        _mlp_body,
        grid=(grid,),
        in_specs=[
            tok_spec(hidden),
            fixed_spec(hidden, rhid),
            fixed_spec(1, rhid),
            fixed_spec(rhid, rhid),
            fixed_spec(1, rhid),
            fixed_spec(rhid, n_e),
        ],
        out_specs=(
            tr_spec(n_e),
            tok_spec(rhid),
        ),
        out_shape=(
            jax.ShapeDtypeStruct((n_e, ntok), jnp.float32),
            jax.ShapeDtypeStruct((ntok, rhid), jnp.float32),
        ),
        compiler_params=pltpu.CompilerParams(
            dimension_semantics=("parallel",)),
    )(x, W1, b1.reshape(1, -1), W2, b2.reshape(1, -1), Wg)

    pt, kit, kpt = _make_sc_router(ntok, n_e)(lt)
    return lt.T, pt.T, kit.T, kpt.T, feat


# speedup vs baseline: 1.2909x; 1.2909x over previous
import jax
import jax.numpy as jnp
from jax.experimental import pallas as pl
from jax.experimental.pallas import tpu as pltpu

T_TILE = 2048


def _router_body(x_ref, w1_ref, b1_ref, w2_ref, b2_ref,
                 wg_ref, logits_ref, probs_ref, tki_ref, tkp_ref, feat_ref):
    h = jnp.maximum(
        jnp.dot(x_ref[...], w1_ref[...], preferred_element_type=jnp.float32)
        + b1_ref[...], 0.0)
    h = jnp.maximum(
        jnp.dot(h, w2_ref[...], preferred_element_type=jnp.float32)
        + b2_ref[...], 0.0)
    feat_ref[...] = h
    logits = jax.lax.dot_general(
        wg_ref[...], h, (((0,), (1,)), ((), ())),
        preferred_element_type=jnp.float32)
    logits_ref[...] = logits

    m = jnp.max(logits, axis=0, keepdims=True)
    e = jnp.exp(logits - m)
    s = jnp.sum(e, axis=0, keepdims=True)
    probs_ref[...] = e / s

    n_e, t = logits.shape
    iota = jax.lax.broadcasted_iota(jnp.int32, (n_e, t), 0)
    pow2 = (1 << iota).astype(jnp.float32)
    eq1 = (logits == m).astype(jnp.float32)
    bits1 = jnp.sum(eq1 * pow2, axis=0, keepdims=True).astype(jnp.int32)
    lsb1 = bits1 & (-bits1)
    i1 = (jax.lax.bitcast_convert_type(lsb1.astype(jnp.float32), jnp.int32)
          >> 23) - 127
    masked = jnp.where(iota == i1, -jnp.inf, logits)
    m2 = jnp.max(masked, axis=0, keepdims=True)
    eq2 = (masked == m2).astype(jnp.float32)
    bits2 = jnp.sum(eq2 * pow2, axis=0, keepdims=True).astype(jnp.int32)
    lsb2 = bits2 & (-bits2)
    i2 = (jax.lax.bitcast_convert_type(lsb2.astype(jnp.float32), jnp.int32)
          >> 23) - 127
    tki_ref[...] = jnp.concatenate([i1, i2], axis=0)
    p1 = 1.0 / s
    p2 = jnp.exp(m2 - m) / s
    denom = p1 + p2
    tkp_ref[...] = jnp.concatenate([p1 / denom, p2 / denom], axis=0)


@jax.jit
def kernel(x, W1, b1, W2, b2, Wg):
    ntok, hidden = x.shape
    rhid = W1.shape[1]
    n_e = Wg.shape[1]
    grid = ntok // T_TILE

    out_shapes = (
        jax.ShapeDtypeStruct((n_e, ntok), jnp.float32),
        jax.ShapeDtypeStruct((n_e, ntok), jnp.float32),
        jax.ShapeDtypeStruct((2, ntok), jnp.int32),
        jax.ShapeDtypeStruct((2, ntok), jnp.float32),
        jax.ShapeDtypeStruct((ntok, rhid), jnp.float32),
    )
    tok_spec = lambda w: pl.BlockSpec((T_TILE, w), lambda i: (i, 0))
    tr_spec = lambda rows: pl.BlockSpec((rows, T_TILE), lambda i: (0, i))
    fixed_spec = lambda a, b: pl.BlockSpec((a, b), lambda i: (0, 0))

    lt, pt, kit, kpt, feat = pl.pallas_call(
        _router_body,
        grid=(grid,),
        in_specs=[
            tok_spec(hidden),
            fixed_spec(hidden, rhid),
            fixed_spec(1, rhid),
            fixed_spec(rhid, rhid),
            fixed_spec(1, rhid),
            fixed_spec(rhid, n_e),
        ],
        out_specs=(
            tr_spec(n_e),
            tr_spec(n_e),
            tr_spec(2),
            tr_spec(2),
            tok_spec(rhid),
        ),
        out_shape=out_shapes,
        compiler_params=pltpu.CompilerParams(
            dimension_semantics=("parallel",)),
    )(x, W1, b1.reshape(1, -1), W2, b2.reshape(1, -1), Wg)
    return lt.T, pt.T, kit.T, kpt.T, feat


# speedup vs baseline: 1.2918x; 1.0008x over previous
import jax
import jax.numpy as jnp
from jax.experimental import pallas as pl
from jax.experimental.pallas import tpu as pltpu

T_TILE = 2048


def _router_body(x_ref, w1_ref, b1_ref, w2_ref, b2_ref,
                 wg_ref, logits_ref, probs_ref, tki_ref, tkp_ref, feat_ref,
                 h_ref):
    h_ref[...] = jnp.maximum(
        jnp.dot(x_ref[...], w1_ref[...], preferred_element_type=jnp.float32)
        + b1_ref[...], 0.0)
    h = jnp.maximum(
        jnp.dot(h_ref[...], w2_ref[...], preferred_element_type=jnp.float32)
        + b2_ref[...], 0.0)
    feat_ref[...] = h
    logits = jax.lax.dot_general(
        wg_ref[...], h, (((0,), (1,)), ((), ())),
        preferred_element_type=jnp.float32)
    logits_ref[...] = logits

    m = jnp.max(logits, axis=0, keepdims=True)
    e = jnp.exp(logits - m)
    s = jnp.sum(e, axis=0, keepdims=True)
    probs_ref[...] = e / s

    n_e, t = logits.shape
    iota = jax.lax.broadcasted_iota(jnp.int32, (n_e, t), 0)
    pow2 = (1 << iota).astype(jnp.float32)
    eq1 = (logits == m).astype(jnp.float32)
    bits1 = jnp.sum(eq1 * pow2, axis=0, keepdims=True).astype(jnp.int32)
    lsb1 = bits1 & (-bits1)
    i1 = (jax.lax.bitcast_convert_type(lsb1.astype(jnp.float32), jnp.int32)
          >> 23) - 127
    masked = jnp.where(iota == i1, -jnp.inf, logits)
    m2 = jnp.max(masked, axis=0, keepdims=True)
    eq2 = (masked == m2).astype(jnp.float32)
    bits2 = jnp.sum(eq2 * pow2, axis=0, keepdims=True).astype(jnp.int32)
    lsb2 = bits2 & (-bits2)
    i2 = (jax.lax.bitcast_convert_type(lsb2.astype(jnp.float32), jnp.int32)
          >> 23) - 127
    tki_ref[...] = jnp.concatenate([i1, i2], axis=0)
    p1 = 1.0 / s
    p2 = jnp.exp(m2 - m) / s
    denom = p1 + p2
    tkp_ref[...] = jnp.concatenate([p1 / denom, p2 / denom], axis=0)


@jax.jit
def kernel(x, W1, b1, W2, b2, Wg):
    ntok, hidden = x.shape
    rhid = W1.shape[1]
    n_e = Wg.shape[1]
    grid = ntok // T_TILE

    out_shapes = (
        jax.ShapeDtypeStruct((n_e, ntok), jnp.float32),
        jax.ShapeDtypeStruct((n_e, ntok), jnp.float32),
        jax.ShapeDtypeStruct((2, ntok), jnp.int32),
        jax.ShapeDtypeStruct((2, ntok), jnp.float32),
        jax.ShapeDtypeStruct((ntok, rhid), jnp.float32),
    )
    tok_spec = lambda w: pl.BlockSpec((T_TILE, w), lambda i: (i, 0))
    tr_spec = lambda rows: pl.BlockSpec((rows, T_TILE), lambda i: (0, i))
    fixed_spec = lambda a, b: pl.BlockSpec((a, b), lambda i: (0, 0))

    lt, pt, kit, kpt, feat = pl.pallas_call(
        _router_body,
        grid=(grid,),
        in_specs=[
            tok_spec(hidden),
            fixed_spec(hidden, rhid),
            fixed_spec(1, rhid),
            fixed_spec(rhid, rhid),
            fixed_spec(1, rhid),
            fixed_spec(rhid, n_e),
        ],
        out_specs=(
            tr_spec(n_e),
            tr_spec(n_e),
            tr_spec(2),
            tr_spec(2),
            tok_spec(rhid),
        ),
        out_shape=out_shapes,
        scratch_shapes=[pltpu.VMEM((T_TILE, rhid), jnp.float32)],
        compiler_params=pltpu.CompilerParams(
            dimension_semantics=("parallel",)),
    )(x, W1, b1.reshape(1, -1), W2, b2.reshape(1, -1), Wg)
    return lt.T, pt.T, kit.T, kpt.T, feat
